# Initial kernel scaffold; baseline (speedup 1.0000x reference)
import jax
import jax.numpy as jnp
from jax.experimental import pallas as pl


def kernel(x):
    raise NotImplementedError("write your pallas kernel here")


# speedup vs baseline: 1.2570x; 1.2570x over previous
import jax
import jax.numpy as jnp
from jax.experimental import pallas as pl
from jax.experimental.pallas import tpu as pltpu

END_BAND = 64


def _copy_kernel(idx_ref, x_ref, o_ref):
    o_ref[...] = x_ref[...]


def _band_indices(num_bands):
    perm_key = jax.random.key(42)
    return jax.random.permutation(perm_key, num_bands)[:END_BAND].astype(jnp.int32)


def kernel(x):
    num_bands = x.shape[0]
    if num_bands <= END_BAND:
        return x
    indices = _band_indices(num_bands)
    H, W = x.shape[1], x.shape[2]
    grid_spec = pltpu.PrefetchScalarGridSpec(
        num_scalar_prefetch=1,
        grid=(END_BAND,),
        in_specs=[
            pl.BlockSpec((1, H, W), lambda i, idx_ref: (idx_ref[i], 0, 0)),
        ],
        out_specs=pl.BlockSpec((1, H, W), lambda i, idx_ref: (i, 0, 0)),
    )
    return pl.pallas_call(
        _copy_kernel,
        grid_spec=grid_spec,
        out_shape=jax.ShapeDtypeStruct((END_BAND, H, W), x.dtype),
    )(indices, x)
